# baseline (speedup 1.0000x reference)
import functools

import jax
import jax.numpy as jnp
from jax import lax
from jax.experimental import pallas as pl
from jax.experimental.pallas import tpu as pltpu
from jax.experimental.pallas import tpu_sc as plsc

EMBED = 64
LANES = 16
NC, NS = 2, 16
NW = NC * NS
CHUNK = 128
NRING = 4
OPITCH = 136


def _make_kernel(B, S):
    n_units = S * (B // CHUNK)
    per_w = n_units // NW
    kb_per_s = B // CHUNK
    mesh = plsc.VectorSubcoreMesh(core_axis_name="c", subcore_axis_name="s")

    @functools.partial(
        pl.kernel,
        mesh=mesh,
        out_type=jax.ShapeDtypeStruct(
            (S, EMBED // 8, B // CHUNK, 8, CHUNK), jnp.float32),
        compiler_params=pltpu.CompilerParams(
            use_tc_tiling_on_sc=False, needs_layout_passes=False),
        scratch_types=[
            pltpu.VMEM((per_w, CHUNK), jnp.int32),
            *[pltpu.VMEM((CHUNK, EMBED), jnp.float32) for _ in range(NRING)],
            pltpu.VMEM((EMBED, OPITCH), jnp.float32),
            pltpu.VMEM((EMBED, OPITCH), jnp.float32),
            pltpu.VMEM((CHUNK,), jnp.float32),
            pltpu.VMEM((CHUNK,), jnp.float32),
            pltpu.VMEM((EMBED,), jnp.float32),
            pltpu.VMEM((EMBED,), jnp.float32),
            *[pltpu.SemaphoreType.DMA for _ in range(NRING + 2)],
        ],
    )
    def k(ids_hbm, table_hbm, gam_hbm, bet_hbm, out_hbm,
          idx_v, r0, r1, r2, r3, oa_v, ob_v, sa_v, sc_v, gam_v, bet_v,
          g0, g1, g2, g3, osa, osb):
        rows = [r0, r1, r2, r3]
        gsem = [g0, g1, g2, g3]
        outs = [oa_v, ob_v]
        osem = [osa, osb]
        wid = lax.axis_index("s") * NC + lax.axis_index("c")
        pltpu.sync_copy(gam_hbm, gam_v)
        pltpu.sync_copy(bet_hbm, bet_v)
        pltpu.sync_copy(ids_hbm.at[pl.ds(wid * per_w, per_w)], idx_v)
        lane = lax.iota(jnp.int32, LANES)
        diagb = lax.shift_right_logical(lane, 1)
        cvbs = [((diagb + kb) & 7) * 8 for kb in range(8)]
        cvecs = [q * LANES + lane for q in range(EMBED // LANES)]

        def fire_gather(c, r):
            pltpu.async_copy(table_hbm.at[idx_v.at[c]], rows[r], gsem[r])

        def wait_gather(r):
            pltpu.make_async_copy(table_hbm.at[idx_v.at[0]],
                                  rows[r], gsem[r]).wait()

        def fire_out(c, out_v, sem):
            rho = wid * per_w + c
            s = ((rho // 256) * 8) + (rho % 8)
            kb = (rho // 8) % kb_per_s
            for eb in range(EMBED // 8):
                pltpu.async_copy(
                    out_v.at[pl.ds(eb * 8, 8), pl.ds(0, CHUNK)],
                    out_hbm.at[s, eb, kb], sem)

        def wait_out(out_v, sem):
            for eb in range(EMBED // 8):
                pltpu.make_async_copy(
                    out_v.at[pl.ds(eb * 8, 8), pl.ds(0, CHUNK)],
                    out_hbm.at[0, eb, 0], sem).wait()

        def compute(rows_v, out_v):
            @plsc.parallel_loop(0, CHUNK // LANES)
            def stats_body(t):
                rvec = t * LANES + lane
                ss = [None] * 4
                qq = [None] * 4
                for kb in range(8):
                    for k0 in range(8):
                        kk = kb * 8 + k0
                        x = plsc.load_gather(rows_v, [rvec, cvbs[kb] + k0])
                        r = kk & 3
                        ss[r] = x if ss[r] is None else ss[r] + x
                        qq[r] = x * x if qq[r] is None else qq[r] + x * x
                ssum = (ss[0] + ss[1]) + (ss[2] + ss[3])
                ssq = (qq[0] + qq[1]) + (qq[2] + qq[3])
                mean = ssum * (1.0 / EMBED)
                var = ssq * (1.0 / EMBED) - mean * mean
                v = var + 1e-5
                iv = plsc.bitcast(v, jnp.int32)
                iv = 0x5F3759DF - lax.shift_right_logical(iv, 1)
                y = plsc.bitcast(iv, jnp.float32)
                h = v * 0.5
                y = y * (1.5 - h * y * y)
                y = y * (1.5 - h * y * y)
                sa_v[pl.ds(t * LANES, LANES)] = y
                sc_v[pl.ds(t * LANES, LANES)] = mean * y

            gs = [gam_v[pl.ds(q * LANES, LANES)]
                  for q in range(EMBED // LANES)]
            bs = [bet_v[pl.ds(q * LANES, LANES)]
                  for q in range(EMBED // LANES)]

            @plsc.parallel_loop(0, 1, unroll=1)
            def norm_body(r):
                tbase = r & ~(LANES - 1)
                av = sa_v[pl.ds(tbase, LANES)]
                cv = sc_v[pl.ds(tbase, LANES)]
                ivec = jnp.full((LANES,), r & (LANES - 1), jnp.int32)
                asp = jnp.take_along_axis(av, ivec, axis=0)
                csp = jnp.take_along_axis(cv, ivec, axis=0)
                rvec = jnp.full((LANES,), r, jnp.int32)
                for q in range(EMBED // LANES):
                    x = rows_v[r, pl.ds(q * LANES, LANES)]
                    o = (x * asp - csp) * gs[q] + bs[q]
                    plsc.store_scatter(out_v, [cvecs[q], rvec], o)

        for r in range(NRING - 1):
            fire_gather(r, r)

        def quad_body(i, carry):
            for q in range(NRING):
                c = NRING * i + q
                fire_gather(jnp.minimum(c + NRING - 1, per_w - 1),
                            (q + NRING - 1) % NRING)
                wait_gather(q)
                op = q & 1
                if q < 2:
                    @pl.when(i > 0)
                    def _():
                        wait_out(outs[op], osem[op])
                else:
                    wait_out(outs[op], osem[op])
                compute(rows[q], outs[op])
                fire_out(c, outs[op], osem[op])
            return carry

        lax.fori_loop(0, per_w // NRING, quad_body, None)
        for r in range(NRING - 1):
            wait_gather(r)
        wait_out(oa_v, osa)
        wait_out(ob_v, osb)

    return k


def kernel(input_ids, table, gamma, beta):
    B, S = input_ids.shape
    _, E = table.shape
    assert E == EMBED and B % CHUNK == 0
    assert (S * B // CHUNK) % (NW * NRING) == 0
    ids2 = (input_ids.astype(jnp.int32)
            .reshape(B // CHUNK, CHUNK, S // 8, 8)
            .transpose(2, 0, 3, 1)
            .reshape(S * B // CHUNK, CHUNK))
    tab = lax.optimization_barrier(
        table.astype(jnp.float32).reshape(-1)).reshape(table.shape)
    out5 = _make_kernel(B, S)(ids2, tab,
                              gamma.astype(jnp.float32),
                              beta.astype(jnp.float32))
    out = jnp.transpose(out5, (2, 4, 0, 1, 3)).reshape(B, S, E)
    return out


# speedup vs baseline: 1.1273x; 1.1273x over previous
import functools

import jax
import jax.numpy as jnp
from jax import lax
from jax.experimental import pallas as pl
from jax.experimental.pallas import tpu as pltpu
from jax.experimental.pallas import tpu_sc as plsc

EMBED = 64
LANES = 16
NC, NS = 2, 16
NW = NC * NS
CHUNK = 128
NRING = 4
OPITCH = 136


def _make_kernel(B, S):
    n_units = S * (B // CHUNK)
    per_w = n_units // NW
    kb_per_s = B // CHUNK
    mesh = plsc.VectorSubcoreMesh(core_axis_name="c", subcore_axis_name="s")

    @functools.partial(
        pl.kernel,
        mesh=mesh,
        out_type=jax.ShapeDtypeStruct(
            (S, EMBED // 8, B // CHUNK, 8, CHUNK), jnp.float32),
        compiler_params=pltpu.CompilerParams(
            use_tc_tiling_on_sc=False, needs_layout_passes=False),
        scratch_types=[
            pltpu.VMEM((per_w, CHUNK), jnp.int32),
            *[pltpu.VMEM((CHUNK, EMBED), jnp.float32) for _ in range(NRING)],
            pltpu.VMEM((EMBED, OPITCH), jnp.float32),
            pltpu.VMEM((EMBED, OPITCH), jnp.float32),
            pltpu.VMEM((CHUNK,), jnp.float32),
            pltpu.VMEM((CHUNK,), jnp.float32),
            pltpu.VMEM((EMBED,), jnp.float32),
            pltpu.VMEM((EMBED,), jnp.float32),
            *[pltpu.SemaphoreType.DMA for _ in range(NRING + 2)],
        ],
    )
    def k(ids_hbm, table_hbm, gam_hbm, bet_hbm, out_hbm,
          idx_v, r0, r1, r2, r3, oa_v, ob_v, sa_v, sc_v, gam_v, bet_v,
          g0, g1, g2, g3, osa, osb):
        rows = [r0, r1, r2, r3]
        gsem = [g0, g1, g2, g3]
        outs = [oa_v, ob_v]
        osem = [osa, osb]
        wid = lax.axis_index("s") * NC + lax.axis_index("c")
        pltpu.sync_copy(gam_hbm, gam_v)
        pltpu.sync_copy(bet_hbm, bet_v)
        pltpu.sync_copy(ids_hbm.at[pl.ds(wid * per_w, per_w)], idx_v)
        lane = lax.iota(jnp.int32, LANES)
        lane15 = jnp.full((LANES,), LANES - 1, jnp.int32)
        lane0m = lane == 0
        cvecs = [q * LANES + lane for q in range(EMBED // LANES)]

        def fire_gather(c, r):
            pltpu.async_copy(table_hbm.at[idx_v.at[c]], rows[r], gsem[r])

        def wait_gather(r):
            pltpu.make_async_copy(table_hbm.at[idx_v.at[0]],
                                  rows[r], gsem[r]).wait()

        def fire_out(c, out_v, sem):
            rho = wid * per_w + c
            s = ((rho // 256) * 8) + (rho % 8)
            kb = (rho // 8) % kb_per_s
            for eb in range(EMBED // 8):
                pltpu.async_copy(
                    out_v.at[pl.ds(eb * 8, 8), pl.ds(0, CHUNK)],
                    out_hbm.at[s, eb, kb], sem)

        def wait_out(out_v, sem):
            for eb in range(EMBED // 8):
                pltpu.make_async_copy(
                    out_v.at[pl.ds(eb * 8, 8), pl.ds(0, CHUNK)],
                    out_hbm.at[0, eb, 0], sem).wait()

        def compute(rows_v, out_v):
            @plsc.parallel_loop(0, CHUNK, unroll=2)
            def sums_body(r):
                xs = [rows_v[r, pl.ds(q * LANES, LANES)]
                      for q in range(EMBED // LANES)]
                s = (xs[0] + xs[1]) + (xs[2] + xs[3])
                sq = ((xs[0] * xs[0] + xs[1] * xs[1])
                      + (xs[2] * xs[2] + xs[3] * xs[3]))
                tot = jnp.take_along_axis(plsc.cumsum(s), lane15, axis=0)
                tot2 = jnp.take_along_axis(plsc.cumsum(sq), lane15, axis=0)
                rvec = jnp.full((LANES,), r, jnp.int32)
                plsc.store_scatter(sa_v, [rvec], tot, mask=lane0m)
                plsc.store_scatter(sc_v, [rvec], tot2, mask=lane0m)

            @plsc.parallel_loop(0, CHUNK // LANES)
            def stats_body(t):
                ssum = sa_v[pl.ds(t * LANES, LANES)]
                ssq = sc_v[pl.ds(t * LANES, LANES)]
                mean = ssum * (1.0 / EMBED)
                var = ssq * (1.0 / EMBED) - mean * mean
                v = var + 1e-5
                iv = plsc.bitcast(v, jnp.int32)
                iv = 0x5F3759DF - lax.shift_right_logical(iv, 1)
                y = plsc.bitcast(iv, jnp.float32)
                h = v * 0.5
                y = y * (1.5 - h * y * y)
                y = y * (1.5 - h * y * y)
                sa_v[pl.ds(t * LANES, LANES)] = y
                sc_v[pl.ds(t * LANES, LANES)] = mean * y

            gs = [gam_v[pl.ds(q * LANES, LANES)]
                  for q in range(EMBED // LANES)]
            bs = [bet_v[pl.ds(q * LANES, LANES)]
                  for q in range(EMBED // LANES)]

            @plsc.parallel_loop(0, CHUNK, unroll=4)
            def norm_body(r):
                tbase = r & ~(LANES - 1)
                av = sa_v[pl.ds(tbase, LANES)]
                cv = sc_v[pl.ds(tbase, LANES)]
                ivec = jnp.full((LANES,), r & (LANES - 1), jnp.int32)
                asp = jnp.take_along_axis(av, ivec, axis=0)
                csp = jnp.take_along_axis(cv, ivec, axis=0)
                rvec = jnp.full((LANES,), r, jnp.int32)
                for q in range(EMBED // LANES):
                    x = rows_v[r, pl.ds(q * LANES, LANES)]
                    o = (x * asp - csp) * gs[q] + bs[q]
                    plsc.store_scatter(out_v, [cvecs[q], rvec], o)

        for r in range(NRING - 1):
            fire_gather(r, r)

        def quad_body(i, carry):
            for q in range(NRING):
                c = NRING * i + q
                fire_gather(jnp.minimum(c + NRING - 1, per_w - 1),
                            (q + NRING - 1) % NRING)
                wait_gather(q)
                op = q & 1
                if q < 2:
                    @pl.when(i > 0)
                    def _():
                        wait_out(outs[op], osem[op])
                else:
                    wait_out(outs[op], osem[op])
                compute(rows[q], outs[op])
                fire_out(c, outs[op], osem[op])
            return carry

        lax.fori_loop(0, per_w // NRING, quad_body, None)
        for r in range(NRING - 1):
            wait_gather(r)
        wait_out(oa_v, osa)
        wait_out(ob_v, osb)

    return k


def kernel(input_ids, table, gamma, beta):
    B, S = input_ids.shape
    _, E = table.shape
    assert E == EMBED and B % CHUNK == 0
    assert (S * B // CHUNK) % (NW * NRING) == 0
    ids2 = (input_ids.astype(jnp.int32)
            .reshape(B // CHUNK, CHUNK, S // 8, 8)
            .transpose(2, 0, 3, 1)
            .reshape(S * B // CHUNK, CHUNK))
    tab = lax.optimization_barrier(
        table.astype(jnp.float32).reshape(-1)).reshape(table.shape)
    out5 = _make_kernel(B, S)(ids2, tab,
                              gamma.astype(jnp.float32),
                              beta.astype(jnp.float32))
    out = jnp.transpose(out5, (2, 4, 0, 1, 3)).reshape(B, S, E)
    return out


# speedup vs baseline: 1.2101x; 1.0735x over previous
import functools

import jax
import jax.numpy as jnp
from jax import lax
from jax.experimental import pallas as pl
from jax.experimental.pallas import tpu as pltpu
from jax.experimental.pallas import tpu_sc as plsc

EMBED = 64
LANES = 16
NC, NS = 2, 16
NW = NC * NS
CHUNK = 128
NRING = 4
OPITCH = 136


def _make_kernel(B, S):
    n_units = S * (B // CHUNK)
    per_w = n_units // NW
    kb_per_s = B // CHUNK
    mesh = plsc.VectorSubcoreMesh(core_axis_name="c", subcore_axis_name="s")

    @functools.partial(
        pl.kernel,
        mesh=mesh,
        out_type=jax.ShapeDtypeStruct(
            (S, EMBED // 8, B // CHUNK, 8, CHUNK), jnp.float32),
        compiler_params=pltpu.CompilerParams(
            use_tc_tiling_on_sc=False, needs_layout_passes=False),
        scratch_types=[
            pltpu.VMEM((per_w, CHUNK), jnp.int32),
            *[pltpu.VMEM((CHUNK, 2 * EMBED), jnp.float32)
              for _ in range(NRING)],
            pltpu.VMEM((EMBED, OPITCH), jnp.float32),
            pltpu.VMEM((EMBED, OPITCH), jnp.float32),
            pltpu.VMEM((CHUNK,), jnp.float32),
            pltpu.VMEM((CHUNK,), jnp.float32),
            pltpu.VMEM((EMBED,), jnp.float32),
            pltpu.VMEM((EMBED,), jnp.float32),
            *[pltpu.SemaphoreType.DMA for _ in range(NRING + 2)],
        ],
    )
    def k(ids_hbm, table_hbm, gam_hbm, bet_hbm, out_hbm,
          idx_v, r0, r1, r2, r3, oa_v, ob_v, sa_v, sc_v, gam_v, bet_v,
          g0, g1, g2, g3, osa, osb):
        rows = [r0, r1, r2, r3]
        gsem = [g0, g1, g2, g3]
        outs = [oa_v, ob_v]
        osem = [osa, osb]
        wid = lax.axis_index("s") * NC + lax.axis_index("c")
        pltpu.sync_copy(gam_hbm, gam_v)
        pltpu.sync_copy(bet_hbm, bet_v)
        pltpu.sync_copy(ids_hbm.at[pl.ds(wid * per_w, per_w)], idx_v)
        lane = lax.iota(jnp.int32, LANES)
        lane15 = jnp.full((LANES,), LANES - 1, jnp.int32)
        lane0m = lane == 0
        cvecs = [q * LANES + lane for q in range(EMBED // LANES)]

        def fire_gather(c, r):
            pltpu.async_copy(table_hbm.at[idx_v.at[c]], rows[r], gsem[r])

        def wait_gather(r):
            pltpu.make_async_copy(table_hbm.at[idx_v.at[0]],
                                  rows[r], gsem[r]).wait()

        def fire_out(c, out_v, sem):
            rho = wid * per_w + c
            s = ((rho // 256) * 8) + (rho % 8)
            kb = (rho // 8) % kb_per_s
            for eb in range(EMBED // 8):
                pltpu.async_copy(
                    out_v.at[pl.ds(eb * 8, 8), pl.ds(0, CHUNK)],
                    out_hbm.at[s, eb, kb], sem)

        def wait_out(out_v, sem):
            for eb in range(EMBED // 8):
                pltpu.make_async_copy(
                    out_v.at[pl.ds(eb * 8, 8), pl.ds(0, CHUNK)],
                    out_hbm.at[0, eb, 0], sem).wait()

        def compute(rows_v, out_v):
            @plsc.parallel_loop(0, CHUNK, unroll=2)
            def sums_body(r):
                xs = [rows_v[r, pl.ds(q * LANES, LANES)]
                      for q in range(EMBED // LANES)]
                s = (xs[0] + xs[1]) + (xs[2] + xs[3])
                sq = ((xs[0] * xs[0] + xs[1] * xs[1])
                      + (xs[2] * xs[2] + xs[3] * xs[3]))
                tot = jnp.take_along_axis(plsc.cumsum(s), lane15, axis=0)
                tot2 = jnp.take_along_axis(plsc.cumsum(sq), lane15, axis=0)
                rvec = jnp.full((LANES,), r, jnp.int32)
                plsc.store_scatter(sa_v, [rvec], tot, mask=lane0m)
                plsc.store_scatter(sc_v, [rvec], tot2, mask=lane0m)

            @plsc.parallel_loop(0, CHUNK // LANES)
            def stats_body(t):
                ssum = sa_v[pl.ds(t * LANES, LANES)]
                ssq = sc_v[pl.ds(t * LANES, LANES)]
                mean = ssum * (1.0 / EMBED)
                var = ssq * (1.0 / EMBED) - mean * mean
                v = var + 1e-5
                iv = plsc.bitcast(v, jnp.int32)
                iv = 0x5F3759DF - lax.shift_right_logical(iv, 1)
                y = plsc.bitcast(iv, jnp.float32)
                h = v * 0.5
                y = y * (1.5 - h * y * y)
                y = y * (1.5 - h * y * y)
                sa_v[pl.ds(t * LANES, LANES)] = y
                sc_v[pl.ds(t * LANES, LANES)] = mean * y

            gs = [gam_v[pl.ds(q * LANES, LANES)]
                  for q in range(EMBED // LANES)]
            bs = [bet_v[pl.ds(q * LANES, LANES)]
                  for q in range(EMBED // LANES)]

            @plsc.parallel_loop(0, CHUNK, unroll=4)
            def norm_body(r):
                tbase = r & ~(LANES - 1)
                av = sa_v[pl.ds(tbase, LANES)]
                cv = sc_v[pl.ds(tbase, LANES)]
                ivec = jnp.full((LANES,), r & (LANES - 1), jnp.int32)
                asp = jnp.take_along_axis(av, ivec, axis=0)
                csp = jnp.take_along_axis(cv, ivec, axis=0)
                rvec = jnp.full((LANES,), r, jnp.int32)
                for q in range(EMBED // LANES):
                    x = rows_v[r, pl.ds(q * LANES, LANES)]
                    o = (x * asp - csp) * gs[q] + bs[q]
                    plsc.store_scatter(out_v, [cvecs[q], rvec], o)

        for r in range(NRING - 1):
            fire_gather(r, r)

        def quad_body(i, carry):
            for q in range(NRING):
                c = NRING * i + q
                fire_gather(jnp.minimum(c + NRING - 1, per_w - 1),
                            (q + NRING - 1) % NRING)
                wait_gather(q)
                op = q & 1
                if q < 2:
                    @pl.when(i > 0)
                    def _():
                        wait_out(outs[op], osem[op])
                else:
                    wait_out(outs[op], osem[op])
                compute(rows[q], outs[op])
                fire_out(c, outs[op], osem[op])
            return carry

        lax.fori_loop(0, per_w // NRING, quad_body, None)
        for r in range(NRING - 1):
            wait_gather(r)
        wait_out(oa_v, osa)
        wait_out(ob_v, osb)

    return k


def kernel(input_ids, table, gamma, beta):
    B, S = input_ids.shape
    _, E = table.shape
    assert E == EMBED and B % CHUNK == 0
    assert (S * B // CHUNK) % (NW * NRING) == 0
    ids2 = (input_ids.astype(jnp.int32)
            .reshape(B // CHUNK, CHUNK, S // 8, 8)
            .transpose(2, 0, 3, 1)
            .reshape(S * B // CHUNK, CHUNK))
    tab = jnp.pad(table.astype(jnp.float32), ((0, 0), (0, EMBED)))
    out5 = _make_kernel(B, S)(ids2, tab,
                              gamma.astype(jnp.float32),
                              beta.astype(jnp.float32))
    out = jnp.transpose(out5, (2, 4, 0, 1, 3)).reshape(B, S, E)
    return out
